# baseline (speedup 1.0000x reference)
import functools

import jax
import jax.numpy as jnp
from jax import lax
from jax.experimental import pallas as pl
from jax.experimental.pallas import tpu as pltpu
from jax.experimental.pallas import tpu_sc as plsc

N = 10000
E = 320000
D = 128
NUM_GRAPHS = 8
P_COUNT = 5
P_DIMS = 256

NC = 2
NS = 16
LANES = 16

EDGES_PER_TILE = E // (NC * NS)
CHUNK = 80
NCHUNK = EDGES_PER_TILE // CHUNK
NPAD = 10240
ROWS_PER_TILE = NPAD // NS
ZROWS = 32

_MESH = plsc.VectorSubcoreMesh(core_axis_name="c", subcore_axis_name="s")


def _zero_vmem(buf, rows, cols):
    zv = jnp.zeros((LANES,), jnp.float32)

    def zrow(i, _):
        def zcol(j, __):
            buf[i, pl.ds(j * LANES, LANES)] = zv
            return 0
        return lax.fori_loop(0, cols // LANES, zcol, 0)

    lax.fori_loop(0, rows, zrow, 0)


COLS_PER_TILE = NPAD // NS


DEG_GRP = 5


@functools.partial(
    pl.kernel,
    mesh=_MESH,
    out_type=jax.ShapeDtypeStruct((NC, NPAD), jnp.float32),
    scratch_types=[
        pltpu.VMEM((NCHUNK, CHUNK), jnp.int32),
        pltpu.VMEM((CHUNK,), jnp.float32),
        pltpu.VMEM((COLS_PER_TILE,), jnp.float32),
        pltpu.VMEM_SHARED((NPAD,), jnp.float32),
        pltpu.SemaphoreType.DMA,
    ],
)
def _deg_kernel(dst_hbm, out_hbm, dst_v, ones_v, zbuf_v, acc_sh, sem):
    c = lax.axis_index("c")
    s = lax.axis_index("s")
    w = c * NS + s

    pltpu.sync_copy(dst_hbm.at[w], dst_v)

    zv = jnp.zeros((LANES,), jnp.float32)
    ov = jnp.ones((LANES,), jnp.float32)

    def zfill(i, _):
        zbuf_v[pl.ds(i * LANES, LANES)] = zv
        return 0
    lax.fori_loop(0, COLS_PER_TILE // LANES, zfill, 0)

    def ofill(i, _):
        ones_v[pl.ds(i * LANES, LANES)] = ov
        return 0
    lax.fori_loop(0, CHUNK // LANES, ofill, 0)

    pltpu.sync_copy(zbuf_v,
                    acc_sh.at[pl.ds(s * COLS_PER_TILE, COLS_PER_TILE)])
    plsc.subcore_barrier()

    def group(j, _):
        handles = [
            pltpu.async_copy(ones_v, acc_sh.at[dst_v.at[j * DEG_GRP + b]],
                             sem, add=True)
            for b in range(DEG_GRP)
        ]
        for h in handles:
            h.wait()
        return 0
    lax.fori_loop(0, NCHUNK // DEG_GRP, group, 0)

    plsc.subcore_barrier()
    pltpu.sync_copy(acc_sh.at[pl.ds(s * COLS_PER_TILE, COLS_PER_TILE)],
                    out_hbm.at[c, pl.ds(s * COLS_PER_TILE, COLS_PER_TILE)])


SCHUNK = 128
SNFULL = EDGES_PER_TILE // SCHUNK
STAIL = EDGES_PER_TILE - SNFULL * SCHUNK
SNBUF = 2


@functools.partial(
    pl.kernel,
    mesh=_MESH,
    out_type=jax.ShapeDtypeStruct((NC, NPAD, D), jnp.float32),
    scratch_types=[
        pltpu.VMEM((EDGES_PER_TILE,), jnp.int32),
        pltpu.VMEM((SNBUF, SCHUNK), jnp.int32),
        pltpu.VMEM((SNBUF, SCHUNK, D), jnp.float32),
        pltpu.VMEM_SHARED((NPAD, D), jnp.float32),
        pltpu.SemaphoreType.DMA,
        pltpu.SemaphoreType.DMA,
    ],
)
def _edge_scatter(src_hbm, dst_hbm, y_hbm, out_hbm,
                  src_v, dstb_v, rows_v, acc_sh, sem0, sem1):
    sems = (sem0, sem1)
    c = lax.axis_index("c")
    s = lax.axis_index("s")
    w = c * NS + s
    ebase = w * EDGES_PER_TILE

    pltpu.sync_copy(src_hbm.at[pl.ds(ebase, EDGES_PER_TILE)], src_v)

    zv = jnp.zeros((LANES,), jnp.float32)

    def zrow(i, _):
        def zcol(j, __):
            rows_v[0, i, pl.ds(j * LANES, LANES)] = zv
            return 0
        return lax.fori_loop(0, D // LANES, zcol, 0)
    lax.fori_loop(0, SCHUNK, zrow, 0)

    def zstripe(r, _):
        pltpu.sync_copy(
            rows_v.at[0],
            acc_sh.at[pl.ds(s * ROWS_PER_TILE + r * SCHUNK, SCHUNK)])
        return 0
    lax.fori_loop(0, ROWS_PER_TILE // SCHUNK, zstripe, 0)
    plsc.subcore_barrier()

    def _issue(i, b):
        off = pl.multiple_of(ebase + i * SCHUNK, 8)
        pltpu.async_copy(dst_hbm.at[pl.ds(off, SCHUNK)], dstb_v.at[b], sems[b])
        pltpu.async_copy(y_hbm.at[src_v.at[pl.ds(i * SCHUNK, SCHUNK)]],
                         rows_v.at[b], sems[b])

    def _drain(i, b):
        off = pl.multiple_of(ebase + i * SCHUNK, 8)
        pltpu.make_async_copy(dst_hbm.at[pl.ds(off, SCHUNK)], dstb_v.at[b],
                              sems[b]).wait()
        pltpu.make_async_copy(y_hbm.at[src_v.at[pl.ds(i * SCHUNK, SCHUNK)]],
                              rows_v.at[b], sems[b]).wait()
        pltpu.sync_copy(rows_v.at[b], acc_sh.at[dstb_v.at[b]], add=True)

    for b in range(SNBUF):
        _issue(b, b)

    def group(j, _):
        for b in range(SNBUF):
            i = j * SNBUF + b
            _drain(i, b)
            nxt = i + SNBUF

            @pl.when(nxt < SNFULL)
            def _prefetch():
                _issue(nxt, b)
        return 0
    lax.fori_loop(0, SNFULL // SNBUF, group, 0)

    toff = pl.multiple_of(ebase + SNFULL * SCHUNK, 8)
    pltpu.sync_copy(dst_hbm.at[pl.ds(toff, STAIL)],
                    dstb_v.at[0, pl.ds(0, STAIL)])
    pltpu.sync_copy(y_hbm.at[src_v.at[pl.ds(SNFULL * SCHUNK, STAIL)]],
                    rows_v.at[0, pl.ds(0, STAIL)])
    pltpu.sync_copy(rows_v.at[0, pl.ds(0, STAIL)],
                    acc_sh.at[dstb_v.at[0, pl.ds(0, STAIL)]], add=True)

    plsc.subcore_barrier()
    pltpu.sync_copy(
        acc_sh.at[pl.ds(s * ROWS_PER_TILE, ROWS_PER_TILE)],
        out_hbm.at[c, pl.ds(s * ROWS_PER_TILE, ROWS_PER_TILE)],
    )


BLK = 1000


def _dinv_from_deg(deg_blk):
    deg = deg_blk[:, :1] + deg_blk[:, 1:2] + 1.0
    return lax.rsqrt(deg)


def _mm1_body(x_ref, deg_ref, w_ref, out_ref):
    dinv = _dinv_from_deg(deg_ref[...])
    xw = jnp.dot(x_ref[...], w_ref[...], preferred_element_type=jnp.float32)
    out_ref[...] = xw * dinv


def _mm1(x, deg2, W1):
    return pl.pallas_call(
        _mm1_body,
        grid=(N // BLK,),
        in_specs=[
            pl.BlockSpec((BLK, D), lambda i: (i, 0)),
            pl.BlockSpec((BLK, 2), lambda i: (i, 0)),
            pl.BlockSpec((D, D), lambda i: (0, 0)),
        ],
        out_specs=pl.BlockSpec((BLK, D), lambda i: (i, 0)),
        out_shape=jax.ShapeDtypeStruct((N, D), jnp.float32),
    )(x, deg2, W1)


def _mm2_body(acc_ref, deg_ref, y1_ref, b1_ref, w2_ref, out_ref):
    dinv = _dinv_from_deg(deg_ref[...])
    acc = acc_ref[0] + acc_ref[1]
    h1 = jnp.maximum(dinv * (acc + y1_ref[...]) + b1_ref[...], 0.0)
    y2 = jnp.dot(h1, w2_ref[...], preferred_element_type=jnp.float32)
    out_ref[...] = y2 * dinv


def _mm2(acc1, deg2, y1, b1, W2):
    return pl.pallas_call(
        _mm2_body,
        grid=(N // BLK,),
        in_specs=[
            pl.BlockSpec((NC, BLK, D), lambda i: (0, i, 0)),
            pl.BlockSpec((BLK, 2), lambda i: (i, 0)),
            pl.BlockSpec((BLK, D), lambda i: (i, 0)),
            pl.BlockSpec((1, D), lambda i: (0, 0)),
            pl.BlockSpec((D, D), lambda i: (0, 0)),
        ],
        out_specs=pl.BlockSpec((BLK, D), lambda i: (i, 0)),
        out_shape=jax.ShapeDtypeStruct((N, D), jnp.float32),
    )(acc1, deg2, y1, b1, W2)


def _final_body(acc_ref, deg_ref, y2_ref, b2_ref, batch_ref,
                wg_ref, bg_ref, wp_ref, bp_ref, wpol_ref, bpol_ref,
                goal_ref, pe_ref, pol_ref, sums_ref, cnt_ref):
    pid = pl.program_id(0)

    @pl.when(pid == 0)
    def _init():
        sums_ref[...] = jnp.zeros_like(sums_ref)
        cnt_ref[...] = jnp.zeros_like(cnt_ref)

    dinv = _dinv_from_deg(deg_ref[...])
    acc = acc_ref[0] + acc_ref[1]
    h2 = jnp.maximum(dinv * (acc + y2_ref[...]) + b2_ref[...], 0.0)

    gids = lax.broadcasted_iota(jnp.int32, (BLK, NUM_GRAPHS), 1)
    onehot = (batch_ref[...] == gids).astype(jnp.float32)
    sums_ref[...] += lax.dot_general(
        onehot, h2, (((0,), (0,)), ((), ())),
        preferred_element_type=jnp.float32)
    cnt1 = jnp.sum(onehot, axis=0)
    cnt_ref[...] += jnp.broadcast_to(cnt1[:, None], (NUM_GRAPHS, D))

    @pl.when(pid == (N // BLK) - 1)
    def _emit():
        pooled = sums_ref[...] / jnp.maximum(cnt_ref[...], 1.0)
        goal_ref[...] = jax.nn.sigmoid(
            jnp.dot(pooled, wg_ref[...], preferred_element_type=jnp.float32)
            + bg_ref[...])
        pe_ref[...] = jnp.dot(
            pooled, wp_ref[...], preferred_element_type=jnp.float32) + bp_ref[...]
        pol_ref[...] = jnp.dot(
            pooled, wpol_ref[...], preferred_element_type=jnp.float32) + bpol_ref[...]


def _final(acc2, deg2, y2, b2, batch2d, Wg, bg, Wp, bp, Wpol, bpol):
    g_dims = Wg.shape[1]
    p_tot = Wp.shape[1]
    pol_n = Wpol.shape[1]
    return pl.pallas_call(
        _final_body,
        grid=(N // BLK,),
        in_specs=[
            pl.BlockSpec((NC, BLK, D), lambda i: (0, i, 0)),
            pl.BlockSpec((BLK, 2), lambda i: (i, 0)),
            pl.BlockSpec((BLK, D), lambda i: (i, 0)),
            pl.BlockSpec((1, D), lambda i: (0, 0)),
            pl.BlockSpec((BLK, 1), lambda i: (i, 0)),
            pl.BlockSpec((D, g_dims), lambda i: (0, 0)),
            pl.BlockSpec((1, g_dims), lambda i: (0, 0)),
            pl.BlockSpec((D, p_tot), lambda i: (0, 0)),
            pl.BlockSpec((1, p_tot), lambda i: (0, 0)),
            pl.BlockSpec((D, pol_n), lambda i: (0, 0)),
            pl.BlockSpec((1, pol_n), lambda i: (0, 0)),
        ],
        out_specs=[
            pl.BlockSpec((NUM_GRAPHS, g_dims), lambda i: (0, 0)),
            pl.BlockSpec((NUM_GRAPHS, p_tot), lambda i: (0, 0)),
            pl.BlockSpec((NUM_GRAPHS, pol_n), lambda i: (0, 0)),
        ],
        out_shape=[
            jax.ShapeDtypeStruct((NUM_GRAPHS, g_dims), jnp.float32),
            jax.ShapeDtypeStruct((NUM_GRAPHS, p_tot), jnp.float32),
            jax.ShapeDtypeStruct((NUM_GRAPHS, pol_n), jnp.float32),
        ],
        scratch_shapes=[
            pltpu.VMEM((NUM_GRAPHS, D), jnp.float32),
            pltpu.VMEM((NUM_GRAPHS, D), jnp.float32),
        ],
    )(acc2, deg2, y2, b2, batch2d, Wg, bg, Wp, bp, Wpol, bpol)


def kernel(x, edge_index, batch, W1, b1, W2, b2, Wg, bg, Wp, bp, Wpol, bpol):
    src = edge_index[0]
    dst = edge_index[1]
    src3 = src.reshape(NC * NS, NCHUNK, CHUNK)
    dst3 = dst.reshape(NC * NS, NCHUNK, CHUNK)
    degc = _deg_kernel(dst3)
    deg2 = degc.T
    y1 = _mm1(x, deg2, W1)
    acc1 = _edge_scatter(src, dst, y1)
    y2 = _mm2(acc1, deg2, y1, b1.reshape(1, -1), W2)
    acc2 = _edge_scatter(src, dst, y2)
    goal, pe, pol = _final(
        acc2, deg2, y2, b2.reshape(1, -1), batch.reshape(-1, 1),
        Wg, bg.reshape(1, -1), Wp, bp.reshape(1, -1), Wpol, bpol.reshape(1, -1))
    return goal, pe.reshape(NUM_GRAPHS, P_COUNT, P_DIMS), pol


# speedup vs baseline: 1.0712x; 1.0712x over previous
import functools

import jax
import jax.numpy as jnp
from jax import lax
from jax.experimental import pallas as pl
from jax.experimental.pallas import tpu as pltpu
from jax.experimental.pallas import tpu_sc as plsc

N = 10000
E = 320000
D = 128
NUM_GRAPHS = 8
P_COUNT = 5
P_DIMS = 256

NC = 2
NS = 16
LANES = 16

EDGES_PER_TILE = E // (NC * NS)
CHUNK = 80
NCHUNK = EDGES_PER_TILE // CHUNK
NPAD = 10240
ROWS_PER_TILE = NPAD // NS
ZROWS = 32

_MESH = plsc.VectorSubcoreMesh(core_axis_name="c", subcore_axis_name="s")


def _zero_vmem(buf, rows, cols):
    zv = jnp.zeros((LANES,), jnp.float32)

    def zrow(i, _):
        def zcol(j, __):
            buf[i, pl.ds(j * LANES, LANES)] = zv
            return 0
        return lax.fori_loop(0, cols // LANES, zcol, 0)

    lax.fori_loop(0, rows, zrow, 0)


COLS_PER_TILE = NPAD // NS


DEG_GRP = 5


@functools.partial(
    pl.kernel,
    mesh=_MESH,
    out_type=jax.ShapeDtypeStruct((NC, NPAD), jnp.float32),
    scratch_types=[
        pltpu.VMEM((NCHUNK, CHUNK), jnp.int32),
        pltpu.VMEM((CHUNK,), jnp.float32),
        pltpu.VMEM((COLS_PER_TILE,), jnp.float32),
        pltpu.VMEM_SHARED((NPAD,), jnp.float32),
        pltpu.SemaphoreType.DMA,
    ],
)
def _deg_kernel(dst_hbm, out_hbm, dst_v, ones_v, zbuf_v, acc_sh, sem):
    c = lax.axis_index("c")
    s = lax.axis_index("s")
    w = c * NS + s

    pltpu.sync_copy(dst_hbm.at[w], dst_v)

    zv = jnp.zeros((LANES,), jnp.float32)
    ov = jnp.ones((LANES,), jnp.float32)

    def zfill(i, _):
        zbuf_v[pl.ds(i * LANES, LANES)] = zv
        return 0
    lax.fori_loop(0, COLS_PER_TILE // LANES, zfill, 0)

    def ofill(i, _):
        ones_v[pl.ds(i * LANES, LANES)] = ov
        return 0
    lax.fori_loop(0, CHUNK // LANES, ofill, 0)

    pltpu.sync_copy(zbuf_v,
                    acc_sh.at[pl.ds(s * COLS_PER_TILE, COLS_PER_TILE)])
    plsc.subcore_barrier()

    def group(j, _):
        handles = [
            pltpu.async_copy(ones_v, acc_sh.at[dst_v.at[j * DEG_GRP + b]],
                             sem, add=True)
            for b in range(DEG_GRP)
        ]
        for h in handles:
            h.wait()
        return 0
    lax.fori_loop(0, NCHUNK // DEG_GRP, group, 0)

    plsc.subcore_barrier()
    pltpu.sync_copy(acc_sh.at[pl.ds(s * COLS_PER_TILE, COLS_PER_TILE)],
                    out_hbm.at[c, pl.ds(s * COLS_PER_TILE, COLS_PER_TILE)])


NBUF = 3


@functools.partial(
    pl.kernel,
    mesh=_MESH,
    out_type=jax.ShapeDtypeStruct((NC, NPAD, D), jnp.float32),
    scratch_types=[
        pltpu.VMEM((EDGES_PER_TILE,), jnp.int32),
        pltpu.VMEM((NBUF, CHUNK), jnp.int32),
        pltpu.VMEM((NBUF, CHUNK, D), jnp.float32),
        pltpu.VMEM((ZROWS, D), jnp.float32),
        pltpu.VMEM_SHARED((NPAD, D), jnp.float32),
        pltpu.SemaphoreType.DMA,
        pltpu.SemaphoreType.DMA,
        pltpu.SemaphoreType.DMA,
    ],
)
def _edge_scatter(src_hbm, dst_hbm, y_hbm, out_hbm,
                  src_v, dstb_v, rows_v, zbuf_v, acc_sh, sem0, sem1, sem2):
    sems = (sem0, sem1, sem2)
    c = lax.axis_index("c")
    s = lax.axis_index("s")
    w = c * NS + s
    ebase = w * EDGES_PER_TILE

    pltpu.sync_copy(src_hbm.at[pl.ds(ebase, EDGES_PER_TILE)], src_v)

    _zero_vmem(zbuf_v, ZROWS, D)

    def zstripe(r, _):
        pltpu.sync_copy(
            zbuf_v, acc_sh.at[pl.ds(s * ROWS_PER_TILE + r * ZROWS, ZROWS)])
        return 0
    lax.fori_loop(0, ROWS_PER_TILE // ZROWS, zstripe, 0)
    plsc.subcore_barrier()

    def _issue(i, b):
        off = pl.multiple_of(ebase + i * CHUNK, 8)
        pltpu.async_copy(dst_hbm.at[pl.ds(off, CHUNK)], dstb_v.at[b], sems[b])
        pltpu.async_copy(y_hbm.at[src_v.at[pl.ds(i * CHUNK, CHUNK)]],
                         rows_v.at[b], sems[b])

    def _drain(i, b):
        off = pl.multiple_of(ebase + i * CHUNK, 8)
        pltpu.make_async_copy(dst_hbm.at[pl.ds(off, CHUNK)], dstb_v.at[b],
                              sems[b]).wait()
        pltpu.make_async_copy(y_hbm.at[src_v.at[pl.ds(i * CHUNK, CHUNK)]],
                              rows_v.at[b], sems[b]).wait()
        pltpu.sync_copy(rows_v.at[b], acc_sh.at[dstb_v.at[b]], add=True)

    for b in range(NBUF):
        _issue(b, b)

    def group(j, _):
        for b in range(NBUF):
            i = j * NBUF + b
            _drain(i, b)
            nxt = i + NBUF

            @pl.when(nxt < NCHUNK)
            def _prefetch():
                _issue(nxt, b)
        return 0
    lax.fori_loop(0, NCHUNK // NBUF, group, 0)

    for b, i in enumerate(range(NBUF * (NCHUNK // NBUF), NCHUNK)):
        _drain(i, b)

    plsc.subcore_barrier()
    pltpu.sync_copy(
        acc_sh.at[pl.ds(s * ROWS_PER_TILE, ROWS_PER_TILE)],
        out_hbm.at[c, pl.ds(s * ROWS_PER_TILE, ROWS_PER_TILE)],
    )


BLK = 1000


def _dinv_from_deg(deg_blk):
    deg = deg_blk[:, :1] + deg_blk[:, 1:2] + 1.0
    return lax.rsqrt(deg)


def _mm1_body(x_ref, deg_ref, w_ref, out_ref):
    dinv = _dinv_from_deg(deg_ref[...])
    xw = jnp.dot(x_ref[...], w_ref[...], preferred_element_type=jnp.float32)
    out_ref[...] = xw * dinv


def _mm1(x, deg2, W1):
    return pl.pallas_call(
        _mm1_body,
        grid=(N // BLK,),
        in_specs=[
            pl.BlockSpec((BLK, D), lambda i: (i, 0)),
            pl.BlockSpec((BLK, 2), lambda i: (i, 0)),
            pl.BlockSpec((D, D), lambda i: (0, 0)),
        ],
        out_specs=pl.BlockSpec((BLK, D), lambda i: (i, 0)),
        out_shape=jax.ShapeDtypeStruct((N, D), jnp.float32),
    )(x, deg2, W1)


def _mm2_body(acc_ref, deg_ref, y1_ref, b1_ref, w2_ref, out_ref):
    dinv = _dinv_from_deg(deg_ref[...])
    acc = acc_ref[0] + acc_ref[1]
    h1 = jnp.maximum(dinv * (acc + y1_ref[...]) + b1_ref[...], 0.0)
    y2 = jnp.dot(h1, w2_ref[...], preferred_element_type=jnp.float32)
    out_ref[...] = y2 * dinv


def _mm2(acc1, deg2, y1, b1, W2):
    return pl.pallas_call(
        _mm2_body,
        grid=(N // BLK,),
        in_specs=[
            pl.BlockSpec((NC, BLK, D), lambda i: (0, i, 0)),
            pl.BlockSpec((BLK, 2), lambda i: (i, 0)),
            pl.BlockSpec((BLK, D), lambda i: (i, 0)),
            pl.BlockSpec((1, D), lambda i: (0, 0)),
            pl.BlockSpec((D, D), lambda i: (0, 0)),
        ],
        out_specs=pl.BlockSpec((BLK, D), lambda i: (i, 0)),
        out_shape=jax.ShapeDtypeStruct((N, D), jnp.float32),
    )(acc1, deg2, y1, b1, W2)


def _final_body(acc_ref, deg_ref, y2_ref, b2_ref, batch_ref,
                wg_ref, bg_ref, wp_ref, bp_ref, wpol_ref, bpol_ref,
                goal_ref, pe_ref, pol_ref, sums_ref, cnt_ref):
    pid = pl.program_id(0)

    @pl.when(pid == 0)
    def _init():
        sums_ref[...] = jnp.zeros_like(sums_ref)
        cnt_ref[...] = jnp.zeros_like(cnt_ref)

    dinv = _dinv_from_deg(deg_ref[...])
    acc = acc_ref[0] + acc_ref[1]
    h2 = jnp.maximum(dinv * (acc + y2_ref[...]) + b2_ref[...], 0.0)

    gids = lax.broadcasted_iota(jnp.int32, (BLK, NUM_GRAPHS), 1)
    onehot = (batch_ref[...] == gids).astype(jnp.float32)
    sums_ref[...] += lax.dot_general(
        onehot, h2, (((0,), (0,)), ((), ())),
        preferred_element_type=jnp.float32)
    cnt1 = jnp.sum(onehot, axis=0)
    cnt_ref[...] += jnp.broadcast_to(cnt1[:, None], (NUM_GRAPHS, D))

    @pl.when(pid == (N // BLK) - 1)
    def _emit():
        pooled = sums_ref[...] / jnp.maximum(cnt_ref[...], 1.0)
        goal_ref[...] = jax.nn.sigmoid(
            jnp.dot(pooled, wg_ref[...], preferred_element_type=jnp.float32)
            + bg_ref[...])
        pe_ref[...] = jnp.dot(
            pooled, wp_ref[...], preferred_element_type=jnp.float32) + bp_ref[...]
        pol_ref[...] = jnp.dot(
            pooled, wpol_ref[...], preferred_element_type=jnp.float32) + bpol_ref[...]


def _final(acc2, deg2, y2, b2, batch2d, Wg, bg, Wp, bp, Wpol, bpol):
    g_dims = Wg.shape[1]
    p_tot = Wp.shape[1]
    pol_n = Wpol.shape[1]
    return pl.pallas_call(
        _final_body,
        grid=(N // BLK,),
        in_specs=[
            pl.BlockSpec((NC, BLK, D), lambda i: (0, i, 0)),
            pl.BlockSpec((BLK, 2), lambda i: (i, 0)),
            pl.BlockSpec((BLK, D), lambda i: (i, 0)),
            pl.BlockSpec((1, D), lambda i: (0, 0)),
            pl.BlockSpec((BLK, 1), lambda i: (i, 0)),
            pl.BlockSpec((D, g_dims), lambda i: (0, 0)),
            pl.BlockSpec((1, g_dims), lambda i: (0, 0)),
            pl.BlockSpec((D, p_tot), lambda i: (0, 0)),
            pl.BlockSpec((1, p_tot), lambda i: (0, 0)),
            pl.BlockSpec((D, pol_n), lambda i: (0, 0)),
            pl.BlockSpec((1, pol_n), lambda i: (0, 0)),
        ],
        out_specs=[
            pl.BlockSpec((NUM_GRAPHS, g_dims), lambda i: (0, 0)),
            pl.BlockSpec((NUM_GRAPHS, p_tot), lambda i: (0, 0)),
            pl.BlockSpec((NUM_GRAPHS, pol_n), lambda i: (0, 0)),
        ],
        out_shape=[
            jax.ShapeDtypeStruct((NUM_GRAPHS, g_dims), jnp.float32),
            jax.ShapeDtypeStruct((NUM_GRAPHS, p_tot), jnp.float32),
            jax.ShapeDtypeStruct((NUM_GRAPHS, pol_n), jnp.float32),
        ],
        scratch_shapes=[
            pltpu.VMEM((NUM_GRAPHS, D), jnp.float32),
            pltpu.VMEM((NUM_GRAPHS, D), jnp.float32),
        ],
    )(acc2, deg2, y2, b2, batch2d, Wg, bg, Wp, bp, Wpol, bpol)


def kernel(x, edge_index, batch, W1, b1, W2, b2, Wg, bg, Wp, bp, Wpol, bpol):
    src = edge_index[0]
    dst = edge_index[1]
    src3 = src.reshape(NC * NS, NCHUNK, CHUNK)
    dst3 = dst.reshape(NC * NS, NCHUNK, CHUNK)
    degc = _deg_kernel(dst3)
    deg2 = degc.T
    y1 = _mm1(x, deg2, W1)
    acc1 = _edge_scatter(src, dst, y1)
    y2 = _mm2(acc1, deg2, y1, b1.reshape(1, -1), W2)
    acc2 = _edge_scatter(src, dst, y2)
    goal, pe, pol = _final(
        acc2, deg2, y2, b2.reshape(1, -1), batch.reshape(-1, 1),
        Wg, bg.reshape(1, -1), Wp, bp.reshape(1, -1), Wpol, bpol.reshape(1, -1))
    return goal, pe.reshape(NUM_GRAPHS, P_COUNT, P_DIMS), pol


# speedup vs baseline: 1.0987x; 1.0257x over previous
import functools

import jax
import jax.numpy as jnp
from jax import lax
from jax.experimental import pallas as pl
from jax.experimental.pallas import tpu as pltpu
from jax.experimental.pallas import tpu_sc as plsc

N = 10000
E = 320000
D = 128
NUM_GRAPHS = 8
P_COUNT = 5
P_DIMS = 256

NC = 2
NS = 16
LANES = 16

EDGES_PER_TILE = E // (NC * NS)
CHUNK = 80
NCHUNK = EDGES_PER_TILE // CHUNK
NPAD = 10240
ROWS_PER_TILE = NPAD // NS
ZROWS = 32

_MESH = plsc.VectorSubcoreMesh(core_axis_name="c", subcore_axis_name="s")


def _zero_vmem(buf, rows, cols):
    zv = jnp.zeros((LANES,), jnp.float32)

    def zrow(i, _):
        def zcol(j, __):
            buf[i, pl.ds(j * LANES, LANES)] = zv
            return 0
        return lax.fori_loop(0, cols // LANES, zcol, 0)

    lax.fori_loop(0, rows, zrow, 0)


COLS_PER_TILE = NPAD // NS


DEG_GRP = 5


@functools.partial(
    pl.kernel,
    mesh=_MESH,
    out_type=jax.ShapeDtypeStruct((NC, NPAD), jnp.float32),
    scratch_types=[
        pltpu.VMEM((NCHUNK, CHUNK), jnp.int32),
        pltpu.VMEM((CHUNK,), jnp.float32),
        pltpu.VMEM((COLS_PER_TILE,), jnp.float32),
        pltpu.VMEM_SHARED((NPAD,), jnp.float32),
        pltpu.SemaphoreType.DMA,
    ],
)
def _deg_kernel(dst_hbm, out_hbm, dst_v, ones_v, zbuf_v, acc_sh, sem):
    c = lax.axis_index("c")
    s = lax.axis_index("s")
    w = c * NS + s

    pltpu.sync_copy(dst_hbm.at[w], dst_v)

    zv = jnp.zeros((LANES,), jnp.float32)
    ov = jnp.ones((LANES,), jnp.float32)

    def zfill(i, _):
        zbuf_v[pl.ds(i * LANES, LANES)] = zv
        return 0
    lax.fori_loop(0, COLS_PER_TILE // LANES, zfill, 0)

    def ofill(i, _):
        ones_v[pl.ds(i * LANES, LANES)] = ov
        return 0
    lax.fori_loop(0, CHUNK // LANES, ofill, 0)

    pltpu.sync_copy(zbuf_v,
                    acc_sh.at[pl.ds(s * COLS_PER_TILE, COLS_PER_TILE)])
    plsc.subcore_barrier()

    def group(j, _):
        handles = [
            pltpu.async_copy(ones_v, acc_sh.at[dst_v.at[j * DEG_GRP + b]],
                             sem, add=True)
            for b in range(DEG_GRP)
        ]
        for h in handles:
            h.wait()
        return 0
    lax.fori_loop(0, NCHUNK // DEG_GRP, group, 0)

    plsc.subcore_barrier()
    pltpu.sync_copy(acc_sh.at[pl.ds(s * COLS_PER_TILE, COLS_PER_TILE)],
                    out_hbm.at[c, pl.ds(s * COLS_PER_TILE, COLS_PER_TILE)])


NBUF = 3


@functools.partial(
    pl.kernel,
    mesh=_MESH,
    out_type=jax.ShapeDtypeStruct((NC, NPAD, D), jnp.float32),
    scratch_types=[
        pltpu.VMEM((EDGES_PER_TILE,), jnp.int32),
        pltpu.VMEM((NBUF, CHUNK), jnp.int32),
        pltpu.VMEM((NBUF, CHUNK, D), jnp.float32),
        pltpu.VMEM((ZROWS, D), jnp.float32),
        pltpu.VMEM_SHARED((NPAD, D), jnp.float32),
        pltpu.SemaphoreType.DMA,
        pltpu.SemaphoreType.DMA,
        pltpu.SemaphoreType.DMA,
    ],
)
def _edge_scatter(src_hbm, dst_hbm, y_hbm, out_hbm,
                  src_v, dstb_v, rows_v, zbuf_v, acc_sh, sem0, sem1, sem2):
    sems = (sem0, sem1, sem2)
    c = lax.axis_index("c")
    s = lax.axis_index("s")
    w = c * NS + s
    ebase = w * EDGES_PER_TILE

    pltpu.sync_copy(src_hbm.at[pl.ds(ebase, EDGES_PER_TILE)], src_v)

    _zero_vmem(zbuf_v, ZROWS, D)

    def zstripe(r, _):
        pltpu.sync_copy(
            zbuf_v, acc_sh.at[pl.ds(s * ROWS_PER_TILE + r * ZROWS, ZROWS)])
        return 0
    lax.fori_loop(0, ROWS_PER_TILE // ZROWS, zstripe, 0)
    plsc.subcore_barrier()

    def _issue(i, b):
        off = pl.multiple_of(ebase + i * CHUNK, 8)
        pltpu.async_copy(dst_hbm.at[pl.ds(off, CHUNK)], dstb_v.at[b], sems[b])
        pltpu.async_copy(y_hbm.at[src_v.at[pl.ds(i * CHUNK, CHUNK)]],
                         rows_v.at[b], sems[b])

    def _drain(i, b):
        off = pl.multiple_of(ebase + i * CHUNK, 8)
        pltpu.make_async_copy(dst_hbm.at[pl.ds(off, CHUNK)], dstb_v.at[b],
                              sems[b]).wait()
        pltpu.make_async_copy(y_hbm.at[src_v.at[pl.ds(i * CHUNK, CHUNK)]],
                              rows_v.at[b], sems[b]).wait()
        pltpu.sync_copy(rows_v.at[b], acc_sh.at[dstb_v.at[b]], add=True)

    for b in range(NBUF):
        _issue(b, b)

    def group(j, _):
        for b in range(NBUF):
            i = j * NBUF + b
            _drain(i, b)
            nxt = i + NBUF

            @pl.when(nxt < NCHUNK)
            def _prefetch():
                _issue(nxt, b)
        return 0
    lax.fori_loop(0, NCHUNK // NBUF, group, 0)

    for b, i in enumerate(range(NBUF * (NCHUNK // NBUF), NCHUNK)):
        _drain(i, b)

    plsc.subcore_barrier()
    pltpu.sync_copy(
        acc_sh.at[pl.ds(s * ROWS_PER_TILE, ROWS_PER_TILE)],
        out_hbm.at[c, pl.ds(s * ROWS_PER_TILE, ROWS_PER_TILE)],
    )


BLK = 2000


def _dinv_from_deg(deg_blk):
    deg = deg_blk[:, :1] + deg_blk[:, 1:2] + 1.0
    return lax.rsqrt(deg)


def _mm1_body(x_ref, deg_ref, w_ref, out_ref):
    dinv = _dinv_from_deg(deg_ref[...])
    xw = jnp.dot(x_ref[...], w_ref[...], preferred_element_type=jnp.float32)
    out_ref[...] = xw * dinv


def _mm1(x, deg2, W1):
    return pl.pallas_call(
        _mm1_body,
        grid=(N // BLK,),
        in_specs=[
            pl.BlockSpec((BLK, D), lambda i: (i, 0)),
            pl.BlockSpec((BLK, 2), lambda i: (i, 0)),
            pl.BlockSpec((D, D), lambda i: (0, 0)),
        ],
        out_specs=pl.BlockSpec((BLK, D), lambda i: (i, 0)),
        out_shape=jax.ShapeDtypeStruct((N, D), jnp.float32),
    )(x, deg2, W1)


def _mm2_body(acc_ref, deg_ref, y1_ref, b1_ref, w2_ref, out_ref):
    dinv = _dinv_from_deg(deg_ref[...])
    acc = acc_ref[0] + acc_ref[1]
    h1 = jnp.maximum(dinv * (acc + y1_ref[...]) + b1_ref[...], 0.0)
    y2 = jnp.dot(h1, w2_ref[...], preferred_element_type=jnp.float32)
    out_ref[...] = y2 * dinv


def _mm2(acc1, deg2, y1, b1, W2):
    return pl.pallas_call(
        _mm2_body,
        grid=(N // BLK,),
        in_specs=[
            pl.BlockSpec((NC, BLK, D), lambda i: (0, i, 0)),
            pl.BlockSpec((BLK, 2), lambda i: (i, 0)),
            pl.BlockSpec((BLK, D), lambda i: (i, 0)),
            pl.BlockSpec((1, D), lambda i: (0, 0)),
            pl.BlockSpec((D, D), lambda i: (0, 0)),
        ],
        out_specs=pl.BlockSpec((BLK, D), lambda i: (i, 0)),
        out_shape=jax.ShapeDtypeStruct((N, D), jnp.float32),
    )(acc1, deg2, y1, b1, W2)


def _final_body(acc_ref, deg_ref, y2_ref, b2_ref, batch_ref,
                wg_ref, bg_ref, wp_ref, bp_ref, wpol_ref, bpol_ref,
                goal_ref, pe_ref, pol_ref, sums_ref, cnt_ref):
    pid = pl.program_id(0)

    @pl.when(pid == 0)
    def _init():
        sums_ref[...] = jnp.zeros_like(sums_ref)
        cnt_ref[...] = jnp.zeros_like(cnt_ref)

    dinv = _dinv_from_deg(deg_ref[...])
    acc = acc_ref[0] + acc_ref[1]
    h2 = jnp.maximum(dinv * (acc + y2_ref[...]) + b2_ref[...], 0.0)

    gids = lax.broadcasted_iota(jnp.int32, (BLK, NUM_GRAPHS), 1)
    onehot = (batch_ref[...] == gids).astype(jnp.float32)
    sums_ref[...] += lax.dot_general(
        onehot, h2, (((0,), (0,)), ((), ())),
        preferred_element_type=jnp.float32)
    cnt1 = jnp.sum(onehot, axis=0)
    cnt_ref[...] += jnp.broadcast_to(cnt1[:, None], (NUM_GRAPHS, D))

    @pl.when(pid == (N // BLK) - 1)
    def _emit():
        pooled = sums_ref[...] / jnp.maximum(cnt_ref[...], 1.0)
        goal_ref[...] = jax.nn.sigmoid(
            jnp.dot(pooled, wg_ref[...], preferred_element_type=jnp.float32)
            + bg_ref[...])
        pe_ref[...] = jnp.dot(
            pooled, wp_ref[...], preferred_element_type=jnp.float32) + bp_ref[...]
        pol_ref[...] = jnp.dot(
            pooled, wpol_ref[...], preferred_element_type=jnp.float32) + bpol_ref[...]


def _final(acc2, deg2, y2, b2, batch2d, Wg, bg, Wp, bp, Wpol, bpol):
    g_dims = Wg.shape[1]
    p_tot = Wp.shape[1]
    pol_n = Wpol.shape[1]
    return pl.pallas_call(
        _final_body,
        grid=(N // BLK,),
        in_specs=[
            pl.BlockSpec((NC, BLK, D), lambda i: (0, i, 0)),
            pl.BlockSpec((BLK, 2), lambda i: (i, 0)),
            pl.BlockSpec((BLK, D), lambda i: (i, 0)),
            pl.BlockSpec((1, D), lambda i: (0, 0)),
            pl.BlockSpec((BLK, 1), lambda i: (i, 0)),
            pl.BlockSpec((D, g_dims), lambda i: (0, 0)),
            pl.BlockSpec((1, g_dims), lambda i: (0, 0)),
            pl.BlockSpec((D, p_tot), lambda i: (0, 0)),
            pl.BlockSpec((1, p_tot), lambda i: (0, 0)),
            pl.BlockSpec((D, pol_n), lambda i: (0, 0)),
            pl.BlockSpec((1, pol_n), lambda i: (0, 0)),
        ],
        out_specs=[
            pl.BlockSpec((NUM_GRAPHS, g_dims), lambda i: (0, 0)),
            pl.BlockSpec((NUM_GRAPHS, p_tot), lambda i: (0, 0)),
            pl.BlockSpec((NUM_GRAPHS, pol_n), lambda i: (0, 0)),
        ],
        out_shape=[
            jax.ShapeDtypeStruct((NUM_GRAPHS, g_dims), jnp.float32),
            jax.ShapeDtypeStruct((NUM_GRAPHS, p_tot), jnp.float32),
            jax.ShapeDtypeStruct((NUM_GRAPHS, pol_n), jnp.float32),
        ],
        scratch_shapes=[
            pltpu.VMEM((NUM_GRAPHS, D), jnp.float32),
            pltpu.VMEM((NUM_GRAPHS, D), jnp.float32),
        ],
    )(acc2, deg2, y2, b2, batch2d, Wg, bg, Wp, bp, Wpol, bpol)


def kernel(x, edge_index, batch, W1, b1, W2, b2, Wg, bg, Wp, bp, Wpol, bpol):
    src = edge_index[0]
    dst = edge_index[1]
    src3 = src.reshape(NC * NS, NCHUNK, CHUNK)
    dst3 = dst.reshape(NC * NS, NCHUNK, CHUNK)
    degc = _deg_kernel(dst3)
    deg2 = degc.T
    y1 = _mm1(x, deg2, W1)
    acc1 = _edge_scatter(src, dst, y1)
    y2 = _mm2(acc1, deg2, y1, b1.reshape(1, -1), W2)
    acc2 = _edge_scatter(src, dst, y2)
    goal, pe, pol = _final(
        acc2, deg2, y2, b2.reshape(1, -1), batch.reshape(-1, 1),
        Wg, bg.reshape(1, -1), Wp, bp.reshape(1, -1), Wpol, bpol.reshape(1, -1))
    return goal, pe.reshape(NUM_GRAPHS, P_COUNT, P_DIMS), pol


# speedup vs baseline: 1.1132x; 1.0132x over previous
import functools

import jax
import jax.numpy as jnp
from jax import lax
from jax.experimental import pallas as pl
from jax.experimental.pallas import tpu as pltpu
from jax.experimental.pallas import tpu_sc as plsc

N = 10000
E = 320000
D = 128
NUM_GRAPHS = 8
P_COUNT = 5
P_DIMS = 256

NC = 2
NS = 16
LANES = 16

EDGES_PER_TILE = E // (NC * NS)
CHUNK = 80
NCHUNK = EDGES_PER_TILE // CHUNK
NPAD = 10240
ROWS_PER_TILE = NPAD // NS
ZROWS = 32

_MESH = plsc.VectorSubcoreMesh(core_axis_name="c", subcore_axis_name="s")


def _zero_vmem(buf, rows, cols):
    zv = jnp.zeros((LANES,), jnp.float32)

    def zrow(i, _):
        def zcol(j, __):
            buf[i, pl.ds(j * LANES, LANES)] = zv
            return 0
        return lax.fori_loop(0, cols // LANES, zcol, 0)

    lax.fori_loop(0, rows, zrow, 0)


COLS_PER_TILE = NPAD // NS


DEG_GRP = 5


@functools.partial(
    pl.kernel,
    mesh=_MESH,
    out_type=jax.ShapeDtypeStruct((NC, NPAD), jnp.float32),
    scratch_types=[
        pltpu.VMEM((NCHUNK, CHUNK), jnp.int32),
        pltpu.VMEM((CHUNK,), jnp.float32),
        pltpu.VMEM((COLS_PER_TILE,), jnp.float32),
        pltpu.VMEM_SHARED((NPAD,), jnp.float32),
        pltpu.SemaphoreType.DMA,
    ],
)
def _deg_kernel(dst_hbm, out_hbm, dst_v, ones_v, zbuf_v, acc_sh, sem):
    c = lax.axis_index("c")
    s = lax.axis_index("s")
    w = c * NS + s

    pltpu.sync_copy(dst_hbm.at[w], dst_v)

    zv = jnp.zeros((LANES,), jnp.float32)
    ov = jnp.ones((LANES,), jnp.float32)

    def zfill(i, _):
        zbuf_v[pl.ds(i * LANES, LANES)] = zv
        return 0
    lax.fori_loop(0, COLS_PER_TILE // LANES, zfill, 0)

    def ofill(i, _):
        ones_v[pl.ds(i * LANES, LANES)] = ov
        return 0
    lax.fori_loop(0, CHUNK // LANES, ofill, 0)

    pltpu.sync_copy(zbuf_v,
                    acc_sh.at[pl.ds(s * COLS_PER_TILE, COLS_PER_TILE)])
    plsc.subcore_barrier()

    def group(j, _):
        handles = [
            pltpu.async_copy(ones_v, acc_sh.at[dst_v.at[j * DEG_GRP + b]],
                             sem, add=True)
            for b in range(DEG_GRP)
        ]
        for h in handles:
            h.wait()
        return 0
    lax.fori_loop(0, NCHUNK // DEG_GRP, group, 0)

    plsc.subcore_barrier()
    pltpu.sync_copy(acc_sh.at[pl.ds(s * COLS_PER_TILE, COLS_PER_TILE)],
                    out_hbm.at[c, pl.ds(s * COLS_PER_TILE, COLS_PER_TILE)])


NBUF = 3


@functools.partial(
    pl.kernel,
    mesh=_MESH,
    out_type=jax.ShapeDtypeStruct((NC, NPAD, D), jnp.float32),
    scratch_types=[
        pltpu.VMEM((EDGES_PER_TILE,), jnp.int32),
        pltpu.VMEM((NBUF, CHUNK), jnp.int32),
        pltpu.VMEM((NBUF, CHUNK, D), jnp.float32),
        pltpu.VMEM((ZROWS, D), jnp.float32),
        pltpu.VMEM_SHARED((NPAD, D), jnp.float32),
        pltpu.SemaphoreType.DMA,
        pltpu.SemaphoreType.DMA,
        pltpu.SemaphoreType.DMA,
    ],
)
def _edge_scatter(src_hbm, dst_hbm, y_hbm, out_hbm,
                  src_v, dstb_v, rows_v, zbuf_v, acc_sh, sem0, sem1, sem2):
    sems = (sem0, sem1, sem2)
    c = lax.axis_index("c")
    s = lax.axis_index("s")
    w = c * NS + s
    ebase = w * EDGES_PER_TILE

    pltpu.sync_copy(src_hbm.at[pl.ds(ebase, EDGES_PER_TILE)], src_v)

    _zero_vmem(zbuf_v, ZROWS, D)

    def zstripe(r, _):
        pltpu.sync_copy(
            zbuf_v, acc_sh.at[pl.ds(s * ROWS_PER_TILE + r * ZROWS, ZROWS)])
        return 0
    lax.fori_loop(0, ROWS_PER_TILE // ZROWS, zstripe, 0)
    plsc.subcore_barrier()

    def _issue(i, b):
        off = pl.multiple_of(ebase + i * CHUNK, 8)
        pltpu.async_copy(dst_hbm.at[pl.ds(off, CHUNK)], dstb_v.at[b], sems[b])
        pltpu.async_copy(y_hbm.at[src_v.at[pl.ds(i * CHUNK, CHUNK)]],
                         rows_v.at[b], sems[b])

    def _drain(i, b):
        off = pl.multiple_of(ebase + i * CHUNK, 8)
        pltpu.make_async_copy(dst_hbm.at[pl.ds(off, CHUNK)], dstb_v.at[b],
                              sems[b]).wait()
        pltpu.make_async_copy(y_hbm.at[src_v.at[pl.ds(i * CHUNK, CHUNK)]],
                              rows_v.at[b], sems[b]).wait()
        pltpu.sync_copy(rows_v.at[b], acc_sh.at[dstb_v.at[b]], add=True)

    for b in range(NBUF):
        _issue(b, b)

    def group(j, _):
        for b in range(NBUF):
            i = j * NBUF + b
            _drain(i, b)
            nxt = i + NBUF

            @pl.when(nxt < NCHUNK)
            def _prefetch():
                _issue(nxt, b)
        return 0
    lax.fori_loop(0, NCHUNK // NBUF, group, 0)

    for b, i in enumerate(range(NBUF * (NCHUNK // NBUF), NCHUNK)):
        _drain(i, b)

    plsc.subcore_barrier()
    pltpu.sync_copy(
        acc_sh.at[pl.ds(s * ROWS_PER_TILE, ROWS_PER_TILE)],
        out_hbm.at[c, pl.ds(s * ROWS_PER_TILE, ROWS_PER_TILE)],
    )


BLK = 5000


def _dinv_from_deg(deg_blk):
    deg = deg_blk[:, :1] + deg_blk[:, 1:2] + 1.0
    return lax.rsqrt(deg)


def _mm1_body(x_ref, deg_ref, w_ref, out_ref):
    dinv = _dinv_from_deg(deg_ref[...])
    xw = jnp.dot(x_ref[...], w_ref[...], preferred_element_type=jnp.float32)
    out_ref[...] = xw * dinv


def _mm1(x, deg2, W1):
    return pl.pallas_call(
        _mm1_body,
        grid=(N // BLK,),
        in_specs=[
            pl.BlockSpec((BLK, D), lambda i: (i, 0)),
            pl.BlockSpec((BLK, 2), lambda i: (i, 0)),
            pl.BlockSpec((D, D), lambda i: (0, 0)),
        ],
        out_specs=pl.BlockSpec((BLK, D), lambda i: (i, 0)),
        out_shape=jax.ShapeDtypeStruct((N, D), jnp.float32),
    )(x, deg2, W1)


def _mm2_body(acc_ref, deg_ref, y1_ref, b1_ref, w2_ref, out_ref):
    dinv = _dinv_from_deg(deg_ref[...])
    acc = acc_ref[0] + acc_ref[1]
    h1 = jnp.maximum(dinv * (acc + y1_ref[...]) + b1_ref[...], 0.0)
    y2 = jnp.dot(h1, w2_ref[...], preferred_element_type=jnp.float32)
    out_ref[...] = y2 * dinv


def _mm2(acc1, deg2, y1, b1, W2):
    return pl.pallas_call(
        _mm2_body,
        grid=(N // BLK,),
        in_specs=[
            pl.BlockSpec((NC, BLK, D), lambda i: (0, i, 0)),
            pl.BlockSpec((BLK, 2), lambda i: (i, 0)),
            pl.BlockSpec((BLK, D), lambda i: (i, 0)),
            pl.BlockSpec((1, D), lambda i: (0, 0)),
            pl.BlockSpec((D, D), lambda i: (0, 0)),
        ],
        out_specs=pl.BlockSpec((BLK, D), lambda i: (i, 0)),
        out_shape=jax.ShapeDtypeStruct((N, D), jnp.float32),
    )(acc1, deg2, y1, b1, W2)


def _final_body(acc_ref, deg_ref, y2_ref, b2_ref, batch_ref,
                wg_ref, bg_ref, wp_ref, bp_ref, wpol_ref, bpol_ref,
                goal_ref, pe_ref, pol_ref, sums_ref, cnt_ref):
    pid = pl.program_id(0)

    @pl.when(pid == 0)
    def _init():
        sums_ref[...] = jnp.zeros_like(sums_ref)
        cnt_ref[...] = jnp.zeros_like(cnt_ref)

    dinv = _dinv_from_deg(deg_ref[...])
    acc = acc_ref[0] + acc_ref[1]
    h2 = jnp.maximum(dinv * (acc + y2_ref[...]) + b2_ref[...], 0.0)

    gids = lax.broadcasted_iota(jnp.int32, (BLK, NUM_GRAPHS), 1)
    onehot = (batch_ref[...] == gids).astype(jnp.float32)
    sums_ref[...] += lax.dot_general(
        onehot, h2, (((0,), (0,)), ((), ())),
        preferred_element_type=jnp.float32)
    cnt1 = jnp.sum(onehot, axis=0)
    cnt_ref[...] += jnp.broadcast_to(cnt1[:, None], (NUM_GRAPHS, D))

    @pl.when(pid == (N // BLK) - 1)
    def _emit():
        pooled = sums_ref[...] / jnp.maximum(cnt_ref[...], 1.0)
        goal_ref[...] = jax.nn.sigmoid(
            jnp.dot(pooled, wg_ref[...], preferred_element_type=jnp.float32)
            + bg_ref[...])
        pe_ref[...] = jnp.dot(
            pooled, wp_ref[...], preferred_element_type=jnp.float32) + bp_ref[...]
        pol_ref[...] = jnp.dot(
            pooled, wpol_ref[...], preferred_element_type=jnp.float32) + bpol_ref[...]


def _final(acc2, deg2, y2, b2, batch2d, Wg, bg, Wp, bp, Wpol, bpol):
    g_dims = Wg.shape[1]
    p_tot = Wp.shape[1]
    pol_n = Wpol.shape[1]
    return pl.pallas_call(
        _final_body,
        grid=(N // BLK,),
        in_specs=[
            pl.BlockSpec((NC, BLK, D), lambda i: (0, i, 0)),
            pl.BlockSpec((BLK, 2), lambda i: (i, 0)),
            pl.BlockSpec((BLK, D), lambda i: (i, 0)),
            pl.BlockSpec((1, D), lambda i: (0, 0)),
            pl.BlockSpec((BLK, 1), lambda i: (i, 0)),
            pl.BlockSpec((D, g_dims), lambda i: (0, 0)),
            pl.BlockSpec((1, g_dims), lambda i: (0, 0)),
            pl.BlockSpec((D, p_tot), lambda i: (0, 0)),
            pl.BlockSpec((1, p_tot), lambda i: (0, 0)),
            pl.BlockSpec((D, pol_n), lambda i: (0, 0)),
            pl.BlockSpec((1, pol_n), lambda i: (0, 0)),
        ],
        out_specs=[
            pl.BlockSpec((NUM_GRAPHS, g_dims), lambda i: (0, 0)),
            pl.BlockSpec((NUM_GRAPHS, p_tot), lambda i: (0, 0)),
            pl.BlockSpec((NUM_GRAPHS, pol_n), lambda i: (0, 0)),
        ],
        out_shape=[
            jax.ShapeDtypeStruct((NUM_GRAPHS, g_dims), jnp.float32),
            jax.ShapeDtypeStruct((NUM_GRAPHS, p_tot), jnp.float32),
            jax.ShapeDtypeStruct((NUM_GRAPHS, pol_n), jnp.float32),
        ],
        scratch_shapes=[
            pltpu.VMEM((NUM_GRAPHS, D), jnp.float32),
            pltpu.VMEM((NUM_GRAPHS, D), jnp.float32),
        ],
    )(acc2, deg2, y2, b2, batch2d, Wg, bg, Wp, bp, Wpol, bpol)


def kernel(x, edge_index, batch, W1, b1, W2, b2, Wg, bg, Wp, bp, Wpol, bpol):
    src = edge_index[0]
    dst = edge_index[1]
    src3 = src.reshape(NC * NS, NCHUNK, CHUNK)
    dst3 = dst.reshape(NC * NS, NCHUNK, CHUNK)
    degc = _deg_kernel(dst3)
    deg2 = degc.T
    y1 = _mm1(x, deg2, W1)
    acc1 = _edge_scatter(src, dst, y1)
    y2 = _mm2(acc1, deg2, y1, b1.reshape(1, -1), W2)
    acc2 = _edge_scatter(src, dst, y2)
    goal, pe, pol = _final(
        acc2, deg2, y2, b2.reshape(1, -1), batch.reshape(-1, 1),
        Wg, bg.reshape(1, -1), Wp, bp.reshape(1, -1), Wpol, bpol.reshape(1, -1))
    return goal, pe.reshape(NUM_GRAPHS, P_COUNT, P_DIMS), pol
